# baseline (speedup 1.0000x reference)
import functools

import jax
import jax.numpy as jnp
from jax import lax
from jax.experimental import pallas as pl
from jax.experimental.pallas import tpu as pltpu
from jax.experimental.pallas import tpu_sc as plsc

N = 10000
E = 320000
D = 128

NC = 2
NS = 16
NW = NC * NS
K = 128


NCHUNK = -(-E // (NW * K))
NCHUNK0 = 63
NCHUNK1 = 2 * NCHUNK - NCHUNK0
E_PAD = NW * NCHUNK * K
N_PAD = N + 112
ROWS_PER_TILE = N_PAD // NS
_CHUNK_SIZES = [128, 128, 128, 128, 120]


def _h1_body(h_ref, w_ref, b_ref, o_ref):
    acc = lax.dot_general(h_ref[...], w_ref[...], (((1,), (1,)), ((), ())),
                          preferred_element_type=jnp.float32)
    o_ref[...] = jnp.maximum(acc + b_ref[...], 0.0)


def _h1(h, W, b2):
    grid = 10
    rb = N // grid
    return pl.pallas_call(
        _h1_body,
        grid=(grid,),
        in_specs=[
            pl.BlockSpec((rb, D), lambda i: (i, 0)),
            pl.BlockSpec((D, D), lambda i: (0, 0)),
            pl.BlockSpec((1, D), lambda i: (0, 0)),
        ],
        out_specs=pl.BlockSpec((rb, D), lambda i: (i, 0)),
        out_shape=jax.ShapeDtypeStruct((N, D), jnp.float32),
    )(h, W, b2)


def _edge_body(h1_hbm, src_hbm, dst_hbm,
               pacc_hbm, pcnt_hbm,
               src_v, dst_v, rows_v, zidx_v, acc_sh, sem):
    cid = lax.axis_index("c")
    sid = lax.axis_index("s")
    r0 = sid * ROWS_PER_TILE
    nchunk = jnp.where(cid == 0, NCHUNK0, NCHUNK1)
    base = jnp.where(cid == 0, sid * NCHUNK0,
                     NS * NCHUNK0 + sid * NCHUNK1)

    def _fill_rows(val):
        def _f(j, _):
            rows_v[j // 8, pl.ds((j % 8) * 16, 16)] = jnp.full((16,), val,
                                                              jnp.float32)
            return 0
        lax.fori_loop(0, K * 8, _f, 0)

    def _idx_chunk(off):
        def _ifill(t, _):
            v = r0 + off + t * 16 + lax.iota(jnp.int32, 16)
            zidx_v[0, pl.ds(t * 16, 16)] = jnp.minimum(v, r0 + ROWS_PER_TILE - 1)
            return 0
        lax.fori_loop(0, K // 16, _ifill, 0)

    def _zero_acc():
        for c in range(len(_CHUNK_SIZES)):
            _idx_chunk(c * K)
            pltpu.sync_copy(rows_v, acc_sh.at[zidx_v.at[0]])

    def _dump_acc(out_hbm):
        off = 0
        for sz in _CHUNK_SIZES:
            _idx_chunk(off)
            pltpu.sync_copy(acc_sh.at[zidx_v.at[0]], rows_v)
            hb = cid * N_PAD + r0 + off
            pltpu.sync_copy(rows_v.at[pl.ds(0, sz)], out_hbm.at[pl.ds(hb, sz)])
            off += sz

    _fill_rows(0.0)
    _zero_acc()
    plsc.subcore_barrier()

    def _chunk_a(j, _):
        row = base + j
        pltpu.sync_copy(src_hbm.at[row], src_v)
        pltpu.sync_copy(dst_hbm.at[row], dst_v)
        pltpu.async_copy(h1_hbm.at[src_v.at[0]], rows_v, sem).wait()
        pltpu.sync_copy(rows_v, acc_sh.at[dst_v.at[0]], add=True)
        return 0
    lax.fori_loop(0, nchunk, _chunk_a, 0)
    plsc.subcore_barrier()
    _dump_acc(pacc_hbm)

    _fill_rows(0.0)
    _zero_acc()
    plsc.subcore_barrier()

    _fill_rows(1.0)
    def _chunk_b(j, _):
        row = base + j
        pltpu.sync_copy(dst_hbm.at[row], dst_v)
        pltpu.sync_copy(rows_v, acc_sh.at[dst_v.at[0]], add=True)
        return 0
    lax.fori_loop(0, nchunk, _chunk_b, 0)
    plsc.subcore_barrier()
    _dump_acc(pcnt_hbm)


_edge_sc = functools.partial(
    pl.kernel,
    out_type=(
        jax.ShapeDtypeStruct((NC * N_PAD, D), jnp.float32),
        jax.ShapeDtypeStruct((NC * N_PAD, D), jnp.float32),
    ),
    mesh=plsc.VectorSubcoreMesh(core_axis_name="c", subcore_axis_name="s",
                                num_cores=NC, num_subcores=NS),
    scratch_types=[
        pltpu.VMEM((1, K), jnp.int32),
        pltpu.VMEM((1, K), jnp.int32),
        pltpu.VMEM((K, D), jnp.float32),
        pltpu.VMEM((1, K), jnp.int32),
        pltpu.VMEM_SHARED((N_PAD, D), jnp.float32),
        pltpu.SemaphoreType.DMA,
    ],
)(_edge_body)


def _fin_body(a0_ref, a1_ref, c0_ref, c1_ref, h1_ref, o_ref):
    sacc = a0_ref[0] + a1_ref[0]
    c = c0_ref[0][:, :1] + c1_ref[0][:, :1]
    mean = sacc / jnp.maximum(c, 1.0)
    o_ref[...] = jnp.where(c > 0, mean, h1_ref[...])


def _finalize(pacc, pcnt, h1):
    grid = 10
    rb = N // grid
    return pl.pallas_call(
        _fin_body,
        grid=(grid,),
        in_specs=[
            pl.BlockSpec((1, rb, D), lambda i: (0, i, 0)),
            pl.BlockSpec((1, rb, D), lambda i: (1, i, 0)),
            pl.BlockSpec((1, rb, D), lambda i: (0, i, 0)),
            pl.BlockSpec((1, rb, D), lambda i: (1, i, 0)),
            pl.BlockSpec((rb, D), lambda i: (i, 0)),
        ],
        out_specs=pl.BlockSpec((rb, D), lambda i: (i, 0)),
        out_shape=jax.ShapeDtypeStruct((N, D), jnp.float32),
    )(pacc, pacc, pcnt, pcnt, h1)


def kernel(h, h_in, edge_index, W, b):
    del h_in
    h1 = _h1(h, W, b.reshape(1, D))

    src = edge_index[0].astype(jnp.int32)
    dst = edge_index[1].astype(jnp.int32)
    pad = E_PAD - E
    src_p = jnp.concatenate([src, jnp.zeros((pad,), jnp.int32)])
    dst_p = jnp.concatenate([dst, jnp.full((pad,), N, jnp.int32)])
    src_p = src_p.reshape(NW * NCHUNK, 1, K)
    dst_p = dst_p.reshape(NW * NCHUNK, 1, K)

    pacc, pcnt = _edge_sc(h1, src_p, dst_p)
    pacc = pacc.reshape(NC, N_PAD, D)
    pcnt = pcnt.reshape(NC, N_PAD, D)
    return _finalize(pacc, pcnt, h1)


# speedup vs baseline: 1.2291x; 1.2291x over previous
import functools

import jax
import jax.numpy as jnp
from jax import lax
from jax.experimental import pallas as pl
from jax.experimental.pallas import tpu as pltpu
from jax.experimental.pallas import tpu_sc as plsc

N = 10000
E = 320000
D = 128

NC = 2
NS = 16
NW = NC * NS
K = 128


NCHUNK = -(-E // (NW * K))
NCHUNK0 = 95
NCHUNK1 = 2 * NCHUNK - NCHUNK0
E_PAD = NW * NCHUNK * K
N_PAD = N + 112
ROWS_PER_TILE = N_PAD // NS
_CHUNK_SIZES = [128, 128, 128, 128, 120]


def _h1_body(h_ref, w_ref, b_ref, o_ref):
    acc = lax.dot_general(h_ref[...], w_ref[...], (((1,), (1,)), ((), ())),
                          preferred_element_type=jnp.float32)
    o_ref[...] = jnp.maximum(acc + b_ref[...], 0.0)


def _h1(h, W, b2):
    grid = 10
    rb = N // grid
    return pl.pallas_call(
        _h1_body,
        grid=(grid,),
        in_specs=[
            pl.BlockSpec((rb, D), lambda i: (i, 0)),
            pl.BlockSpec((D, D), lambda i: (0, 0)),
            pl.BlockSpec((1, D), lambda i: (0, 0)),
        ],
        out_specs=pl.BlockSpec((rb, D), lambda i: (i, 0)),
        out_shape=jax.ShapeDtypeStruct((N, D), jnp.float32),
    )(h, W, b2)


def _edge_body(h1_hbm, src_hbm, dst_hbm,
               pacc_hbm, pcnt_hbm,
               src_v, dst_v, rows_v, zidx_v, acc_sh, sem):
    cid = lax.axis_index("c")
    sid = lax.axis_index("s")
    r0 = sid * ROWS_PER_TILE
    nchunk = jnp.where(cid == 0, NCHUNK0, NCHUNK1)
    base = jnp.where(cid == 0, sid * NCHUNK0,
                     NS * NCHUNK0 + sid * NCHUNK1)

    def _fill_rows(val):
        def _f(j, _):
            rows_v[j // 8, pl.ds((j % 8) * 16, 16)] = jnp.full((16,), val,
                                                              jnp.float32)
            return 0
        lax.fori_loop(0, K * 8, _f, 0)

    def _idx_chunk(off):
        def _ifill(t, _):
            v = r0 + off + t * 16 + lax.iota(jnp.int32, 16)
            zidx_v[0, pl.ds(t * 16, 16)] = jnp.minimum(v, r0 + ROWS_PER_TILE - 1)
            return 0
        lax.fori_loop(0, K // 16, _ifill, 0)

    def _zero_acc():
        for c in range(len(_CHUNK_SIZES)):
            _idx_chunk(c * K)
            pltpu.sync_copy(rows_v, acc_sh.at[zidx_v.at[0]])

    def _dump_acc(out_hbm):
        off = 0
        for sz in _CHUNK_SIZES:
            _idx_chunk(off)
            pltpu.sync_copy(acc_sh.at[zidx_v.at[0]], rows_v)
            hb = cid * N_PAD + r0 + off
            pltpu.sync_copy(rows_v.at[pl.ds(0, sz)], out_hbm.at[pl.ds(hb, sz)])
            off += sz

    _fill_rows(0.0)
    _zero_acc()
    plsc.subcore_barrier()

    def _chunk_a(j, _):
        row = base + j
        pltpu.sync_copy(src_hbm.at[row], src_v)
        pltpu.sync_copy(dst_hbm.at[row], dst_v)
        pltpu.async_copy(h1_hbm.at[src_v.at[0]], rows_v, sem).wait()
        pltpu.sync_copy(rows_v, acc_sh.at[dst_v.at[0]], add=True)
        return 0
    lax.fori_loop(0, nchunk, _chunk_a, 0)
    plsc.subcore_barrier()
    _dump_acc(pacc_hbm)

    _fill_rows(0.0)
    _zero_acc()
    plsc.subcore_barrier()

    _fill_rows(1.0)
    def _chunk_b(j, _):
        row = base + j
        pltpu.sync_copy(dst_hbm.at[row], dst_v)
        pltpu.sync_copy(rows_v, acc_sh.at[dst_v.at[0]], add=True)
        return 0
    lax.fori_loop(0, nchunk, _chunk_b, 0)
    plsc.subcore_barrier()
    _dump_acc(pcnt_hbm)


_edge_sc = functools.partial(
    pl.kernel,
    out_type=(
        jax.ShapeDtypeStruct((NC * N_PAD, D), jnp.float32),
        jax.ShapeDtypeStruct((NC * N_PAD, D), jnp.float32),
    ),
    mesh=plsc.VectorSubcoreMesh(core_axis_name="c", subcore_axis_name="s",
                                num_cores=NC, num_subcores=NS),
    scratch_types=[
        pltpu.VMEM((1, K), jnp.int32),
        pltpu.VMEM((1, K), jnp.int32),
        pltpu.VMEM((K, D), jnp.float32),
        pltpu.VMEM((1, K), jnp.int32),
        pltpu.VMEM_SHARED((N_PAD, D), jnp.float32),
        pltpu.SemaphoreType.DMA,
    ],
)(_edge_body)


def _fin_body(a0_ref, a1_ref, c0_ref, c1_ref, h1_ref, o_ref):
    sacc = a0_ref[0] + a1_ref[0]
    c = c0_ref[0][:, :1] + c1_ref[0][:, :1]
    mean = sacc / jnp.maximum(c, 1.0)
    o_ref[...] = jnp.where(c > 0, mean, h1_ref[...])


def _finalize(pacc, pcnt, h1):
    grid = 10
    rb = N // grid
    return pl.pallas_call(
        _fin_body,
        grid=(grid,),
        in_specs=[
            pl.BlockSpec((1, rb, D), lambda i: (0, i, 0)),
            pl.BlockSpec((1, rb, D), lambda i: (1, i, 0)),
            pl.BlockSpec((1, rb, D), lambda i: (0, i, 0)),
            pl.BlockSpec((1, rb, D), lambda i: (1, i, 0)),
            pl.BlockSpec((rb, D), lambda i: (i, 0)),
        ],
        out_specs=pl.BlockSpec((rb, D), lambda i: (i, 0)),
        out_shape=jax.ShapeDtypeStruct((N, D), jnp.float32),
    )(pacc, pacc, pcnt, pcnt, h1)


def kernel(h, h_in, edge_index, W, b):
    del h_in
    h1 = _h1(h, W, b.reshape(1, D))

    src = edge_index[0].astype(jnp.int32)
    dst = edge_index[1].astype(jnp.int32)
    pad = E_PAD - E
    src_p = jnp.concatenate([src, jnp.zeros((pad,), jnp.int32)])
    dst_p = jnp.concatenate([dst, jnp.full((pad,), N, jnp.int32)])
    src_p = src_p.reshape(NW * NCHUNK, 1, K)
    dst_p = dst_p.reshape(NW * NCHUNK, 1, K)

    pacc, pcnt = _edge_sc(h1, src_p, dst_p)
    pacc = pacc.reshape(NC, N_PAD, D)
    pcnt = pcnt.reshape(NC, N_PAD, D)
    return _finalize(pacc, pcnt, h1)
